# baseline (speedup 1.0000x reference)
import functools

import jax
import jax.numpy as jnp
from jax import lax
from jax.experimental import pallas as pl
from jax.experimental.pallas import tpu as pltpu
from jax.experimental.pallas import tpu_sc as plsc

B = 32
P = 16800
G = 8
L = 16
CHUNK = 1680
NCHUNK = P // CHUNK
GROUPS = CHUNK // L
TGT_STRIDE = 15
THRESHOLD = 0.35
NEGPOS_RATIO = 7
VAR0_INV = 10.0
VAR1_INV = 5.0
LN2 = 0.6931471805599453


def _vlog(x):
    bits = plsc.bitcast(x, jnp.int32)
    e = (bits >> 23) - 127
    m = plsc.bitcast((bits & 0x7FFFFF) | (127 << 23), jnp.float32)
    big = m > 1.4142135623730951
    m = jnp.where(big, m * 0.5, m)
    e = jnp.where(big, e + 1, e)
    s = (m - 1.0) / (m + 1.0)
    z = s * s
    p = 2.0 * s * (1.0 + z * (1.0 / 3.0 + z * (0.2 + z * (1.0 / 7.0 + z * (1.0 / 9.0)))))
    return e.astype(jnp.float32) * LN2 + p


def _vlog1p(e):
    u = e / (e + 2.0)
    z = u * u
    return 2.0 * u * (1.0 + z * (1.0 / 3.0 + z * (0.2 + z * (1.0 / 7.0
                      + z * (1.0 / 9.0 + z * (1.0 / 11.0))))))


def _sl1(a, b):
    d = jnp.abs(a - b)
    return jnp.where(d < 1.0, 0.5 * d * d, d - 0.5)


def _body(cls_h, loc_h, lm_h, pri_h, tgt_h, out_h,
          pbuf, cbuf, lbuf, mbuf, tbuf, bto, bti, lcb, hist, obuf):
    cid = lax.axis_index("c")
    sid = lax.axis_index("s")
    wid = sid * 2 + cid
    b = wid

    pltpu.sync_copy(tgt_h.at[pl.ds(pl.multiple_of(b * (G * TGT_STRIDE), 8),
                                   G * TGT_STRIDE)], tbuf.at[pl.ds(0, G * TGT_STRIDE)])

    idx16 = lax.iota(jnp.int32, 16)
    idx2 = idx16 * 2
    idx4 = idx16 * 4
    idx10 = idx16 * 10

    tvs = [tbuf[pl.ds(g * TGT_STRIDE, 16)] for g in range(G)]
    tx1 = [tvs[g][0] for g in range(G)]
    ty1 = [tvs[g][1] for g in range(G)]
    tx2 = [tvs[g][2] for g in range(G)]
    ty2 = [tvs[g][3] for g in range(G)]
    area_t = [(tx2[g] - tx1[g]) * (ty2[g] - ty1[g]) for g in range(G)]

    def p1_chunk(ci, carry):
        base = pl.multiple_of(ci * CHUNK, CHUNK)
        pltpu.sync_copy(pri_h.at[pl.ds(base * 4, CHUNK * 4)], pbuf)

        def p1_grp(gi, carry):
            bestv = list(carry[0:G])
            besti = list(carry[G:2 * G])
            off = gi * (L * 4)
            px = plsc.load_gather(pbuf, [idx4 + off])
            py = plsc.load_gather(pbuf, [idx4 + (off + 1)])
            pw = plsc.load_gather(pbuf, [idx4 + (off + 2)])
            ph = plsc.load_gather(pbuf, [idx4 + (off + 3)])
            x1 = px - pw * 0.5
            y1 = py - ph * 0.5
            x2 = px + pw * 0.5
            y2 = py + ph * 0.5
            area_p = pw * ph
            start = base + gi * L
            pidx = start + idx16
            btov = None
            btiv = None
            for g in range(G):
                ix = jnp.minimum(x2, tx2[g]) - jnp.maximum(x1, tx1[g])
                iy = jnp.minimum(y2, ty2[g]) - jnp.maximum(y1, ty1[g])
                inter = jnp.maximum(ix, 0.0) * jnp.maximum(iy, 0.0)
                iou = inter / (area_t[g] + area_p - inter)
                if g == 0:
                    btov = iou
                    btiv = jnp.zeros((L,), jnp.int32)
                else:
                    upd = iou > btov
                    btov = jnp.where(upd, iou, btov)
                    btiv = jnp.where(upd, g, btiv)
                gupd = iou > bestv[g]
                bestv[g] = jnp.where(gupd, iou, bestv[g])
                besti[g] = jnp.where(gupd, pidx, besti[g])
            plsc.store_scatter(bto, [pidx], btov)
            plsc.store_scatter(bti, [pidx], btiv)
            return tuple(bestv) + tuple(besti)

        return lax.fori_loop(0, GROUPS, p1_grp, carry)

    init = tuple(jnp.full((L,), -1.0, jnp.float32) for _ in range(G)) + \
           tuple(jnp.zeros((L,), jnp.int32) for _ in range(G))
    carry = lax.fori_loop(0, NCHUNK, p1_chunk, init)
    bestv = carry[0:G]
    besti = carry[G:2 * G]

    bpo = []
    bpi = []
    for g in range(G):
        m = jnp.max(bestv[g])
        cand = jnp.where(bestv[g] == m, besti[g], jnp.int32(P))
        bpo.append(m)
        bpi.append(jnp.min(cand))
    valid = [bpo[g] >= 0.2 for g in range(G)]
    av = valid[0]
    for g in range(1, G):
        av = jnp.logical_or(av, valid[g])

    def p2_chunk(ci, carry):
        base = pl.multiple_of(ci * CHUNK, CHUNK)
        pltpu.sync_copy(pri_h.at[pl.ds(base * 4, CHUNK * 4)], pbuf)
        pltpu.sync_copy(cls_h.at[pl.ds(pl.multiple_of(b * (P * 2) + base * 2, 8),
                                       CHUNK * 2)], cbuf)
        pltpu.sync_copy(loc_h.at[pl.ds(pl.multiple_of(b * (P * 4) + base * 4, 8),
                                       CHUNK * 4)], lbuf)
        pltpu.sync_copy(lm_h.at[pl.ds(pl.multiple_of(b * (P * 10) + base * 10, 8),
                                      CHUNK * 10)], mbuf)

        def p2_grp(gi, carry):
            acc_l, acc_c, acc_m, acc_p = carry
            off2 = gi * (L * 2)
            off4 = gi * (L * 4)
            off10 = gi * (L * 10)
            start = base + gi * L
            pidx = start + idx16
            px = plsc.load_gather(pbuf, [idx4 + off4])
            py = plsc.load_gather(pbuf, [idx4 + (off4 + 1)])
            pw = plsc.load_gather(pbuf, [idx4 + (off4 + 2)])
            ph = plsc.load_gather(pbuf, [idx4 + (off4 + 3)])

            orig = plsc.load_gather(bto, [pidx])
            cur = orig
            curi = plsc.load_gather(bti, [pidx])
            for g in range(G):
                hit = pidx == bpi[g]
                upd = jnp.where(valid[g], jnp.float32(2.0), orig)
                cur = jnp.where(hit, upd, cur)
                curi = jnp.where(hit, jnp.int32(g), curi)

            tb = curi * TGT_STRIDE
            glab = plsc.load_gather(tbuf, [tb + 14])
            pos = jnp.logical_and(cur >= THRESHOLD, glab.astype(jnp.int32) != 0)
            pos = jnp.logical_and(pos, av)
            acc_p = acc_p + jnp.where(pos, 1.0, 0.0)

            mx1 = plsc.load_gather(tbuf, [tb])
            my1 = plsc.load_gather(tbuf, [tb + 1])
            mx2 = plsc.load_gather(tbuf, [tb + 2])
            my2 = plsc.load_gather(tbuf, [tb + 3])
            g_cx = ((mx1 + mx2) * 0.5 - px) / pw * VAR0_INV
            g_cy = ((my1 + my2) * 0.5 - py) / ph * VAR0_INV
            g_w = _vlog((mx2 - mx1) / pw) * VAR1_INV
            g_h = _vlog((my2 - my1) / ph) * VAR1_INV
            l0 = plsc.load_gather(lbuf, [idx4 + off4])
            l1 = plsc.load_gather(lbuf, [idx4 + (off4 + 1)])
            l2 = plsc.load_gather(lbuf, [idx4 + (off4 + 2)])
            l3 = plsc.load_gather(lbuf, [idx4 + (off4 + 3)])
            vloc = _sl1(l0, g_cx) + _sl1(l1, g_cy) + _sl1(l2, g_w) + _sl1(l3, g_h)
            acc_l = acc_l + jnp.where(pos, vloc, 0.0)

            vlm = None
            for j in range(5):
                for xy in range(2):
                    ch = 2 * j + xy
                    mv = plsc.load_gather(tbuf, [tb + (4 + ch)])
                    pc = px if xy == 0 else py
                    ps = pw if xy == 0 else ph
                    tv = (mv - pc) / ps * VAR0_INV
                    dat = plsc.load_gather(mbuf, [idx10 + (off10 + ch)])
                    t = _sl1(dat, tv)
                    vlm = t if vlm is None else vlm + t
            acc_m = acc_m + jnp.where(pos, vlm, 0.0)

            c0 = plsc.load_gather(cbuf, [idx2 + off2])
            c1 = plsc.load_gather(cbuf, [idx2 + (off2 + 1)])
            mx = jnp.maximum(c0, c1)
            lse = mx + _vlog1p(jnp.exp(-jnp.abs(c0 - c1)))
            acc_c = acc_c + jnp.where(pos, lse - c1, 0.0)
            plsc.store_scatter(lcb, [pidx], jnp.where(pos, 0.0, lse - c0))
            return acc_l, acc_c, acc_m, acc_p

        return lax.fori_loop(0, GROUPS, p2_grp, carry)

    zf = jnp.zeros((L,), jnp.float32)
    acc_l, acc_c, acc_m, acc_p = lax.fori_loop(0, NCHUNK, p2_chunk, (zf, zf, zf, zf))

    npos = jnp.sum(acc_p)
    k = jnp.minimum(npos.astype(jnp.int32) * NEGPOS_RATIO, P - 1)

    ones_i = jnp.ones((L,), jnp.int32)
    prefix = jnp.int32(0)
    k_cur = k
    for shift, nbits in ((23, 8), (15, 8), (7, 8), (0, 7)):
        himask = jnp.int32(-(1 << (shift + nbits)))

        def h_zero(gi, _):
            plsc.store_scatter(hist, [gi * L + idx16], jnp.zeros((L,), jnp.int32))
            return 0

        lax.fori_loop(0, 256 // L, h_zero, 0)

        def h_grp(gi, _):
            bits = plsc.bitcast(plsc.load_gather(lcb, [gi * L + idx16]), jnp.int32)
            match = (bits & himask) == prefix
            bucket = (bits >> shift) & ((1 << nbits) - 1)
            plsc.addupdate_scatter(hist, [bucket], ones_i, mask=match)
            return 0

        lax.fori_loop(0, P // L, h_grp, 0)

        bstar = jnp.int32(-1)
        carry_cnt = jnp.int32(0)
        for gi in reversed(range(256 // L)):
            h = hist[pl.ds(gi * L, L)]
            sfx = lax.rev(plsc.cumsum(lax.rev(h, (0,))), (0,))
            ge = sfx + carry_cnt
            gidx = gi * L + idx16
            cand = jnp.where(ge >= k_cur, gidx, jnp.int32(-1))
            bstar = jnp.maximum(bstar, jnp.max(cand))
            carry_cnt = carry_cnt + jnp.max(sfx)
        cgt = jnp.int32(0)
        for gi in range(256 // L):
            h = hist[pl.ds(gi * L, L)]
            gidx = gi * L + idx16
            cgt = cgt + jnp.sum(jnp.where(gidx > bstar, h, 0))
        k_cur = k_cur - cgt
        prefix = prefix | jnp.where(bstar > 0, bstar, 0) << shift

    t_bits = prefix

    def f_grp(gi, carry):
        cgt_a, s_a = carry
        v = plsc.load_gather(lcb, [gi * L + idx16])
        bits = plsc.bitcast(v, jnp.int32)
        gt = bits > t_bits
        return cgt_a + jnp.where(gt, 1, 0), s_a + jnp.where(gt, v, 0.0)

    cgt_a, s_a = lax.fori_loop(0, P // L, f_grp,
                               (jnp.zeros((L,), jnp.int32), zf))
    cgt = jnp.sum(cgt_a)
    s_gt = jnp.sum(s_a)
    t_val = lax.bitcast_convert_type(t_bits, jnp.float32)
    topk = s_gt + (k - cgt).astype(jnp.float32) * t_val
    topk = jnp.where(k > 0, topk, 0.0)

    sum_l = jnp.sum(acc_l)
    sum_c = jnp.sum(acc_c) + topk
    sum_m = jnp.sum(acc_m)
    res = jnp.where(idx16 == 0, sum_l, 0.0)
    res = jnp.where(idx16 == 1, sum_c, res)
    res = jnp.where(idx16 == 2, sum_m, res)
    res = jnp.where(idx16 == 3, npos, res)
    obuf[...] = res
    pltpu.sync_copy(obuf, out_h.at[pl.ds(pl.multiple_of(wid * L, 8), L)])


@functools.partial(jax.jit, static_argnames=())
def _run(cls_f, loc_f, lm_f, pri_f, tgt_f):
    mesh = plsc.VectorSubcoreMesh(core_axis_name="c", subcore_axis_name="s",
                                  num_cores=2, num_subcores=16)
    f = pl.kernel(
        _body,
        out_type=jax.ShapeDtypeStruct((B * L,), jnp.float32),
        mesh=mesh,
        scratch_types=[
            pltpu.VMEM((CHUNK * 4,), jnp.float32),
            pltpu.VMEM((CHUNK * 2,), jnp.float32),
            pltpu.VMEM((CHUNK * 4,), jnp.float32),
            pltpu.VMEM((CHUNK * 10,), jnp.float32),
            pltpu.VMEM((G * TGT_STRIDE + 16,), jnp.float32),
            pltpu.VMEM((P,), jnp.float32),
            pltpu.VMEM((P,), jnp.int32),
            pltpu.VMEM((P,), jnp.float32),
            pltpu.VMEM((256,), jnp.int32),
            pltpu.VMEM((L,), jnp.float32),
        ],
        compiler_params=pltpu.CompilerParams(needs_layout_passes=False),
    )
    return f(cls_f, loc_f, lm_f, pri_f, tgt_f)


def kernel(cls_data, loc_data, landm_data, priors, targets):
    cls_f = cls_data.reshape(B * P * 2)
    loc_f = loc_data.reshape(B * P * 4)
    lm_f = landm_data.reshape(B * P * 10)
    pri_f = priors.reshape(P * 4)
    tgt_f = targets.reshape(B * G * TGT_STRIDE)
    out = _run(cls_f, loc_f, lm_f, pri_f, tgt_f).reshape(B, L)
    npos = jnp.sum(out[:, 3])
    n = jnp.maximum(npos, 1.0)
    return jnp.sum(out[:, 0]) / n, jnp.sum(out[:, 1]) / n, jnp.sum(out[:, 2]) / n


# speedup vs baseline: 3.6614x; 3.6614x over previous
import functools

import jax
import jax.numpy as jnp
from jax import lax
from jax.experimental import pallas as pl
from jax.experimental.pallas import tpu as pltpu
from jax.experimental.pallas import tpu_sc as plsc

B = 32
P = 16800
G = 8
L = 16
CHUNK = 1680
NCHUNK = P // CHUNK
GROUPS = CHUNK // L
TGT_STRIDE = 15
THRESHOLD = 0.35
NEGPOS_RATIO = 7
VAR0_INV = 10.0
VAR1_INV = 5.0
LN2 = 0.6931471805599453


def _vlog(x):
    bits = plsc.bitcast(x, jnp.int32)
    e = (bits >> 23) - 127
    m = plsc.bitcast((bits & 0x7FFFFF) | (127 << 23), jnp.float32)
    big = m > 1.4142135623730951
    m = jnp.where(big, m * 0.5, m)
    e = jnp.where(big, e + 1, e)
    s = (m - 1.0) / (m + 1.0)
    z = s * s
    p = 2.0 * s * (1.0 + z * (1.0 / 3.0 + z * (0.2 + z * (1.0 / 7.0 + z * (1.0 / 9.0)))))
    return e.astype(jnp.float32) * LN2 + p


def _vlog1p(e):
    u = e / (e + 2.0)
    z = u * u
    return 2.0 * u * (1.0 + z * (1.0 / 3.0 + z * (0.2 + z * (1.0 / 7.0
                      + z * (1.0 / 9.0 + z * (1.0 / 11.0))))))


def _sl1(a, b):
    d = jnp.abs(a - b)
    return jnp.where(d < 1.0, 0.5 * d * d, d - 0.5)


def _body(cls_h, loc_h, lm_h, pri_h, tgt_h, out_h,
          pbuf, cbuf, lbuf, mbuf, tbuf, bto, bti, lcb, hist, obuf, dsem):
    cid = lax.axis_index("c")
    sid = lax.axis_index("s")
    wid = sid * 2 + cid
    b = wid

    pltpu.sync_copy(tgt_h.at[b], tbuf)

    idx16 = lax.iota(jnp.int32, 16)
    cl14 = jnp.minimum(idx16, 14)

    tvs = [plsc.load_gather(tbuf, [jnp.full((L,), g, jnp.int32), cl14])
           for g in range(G)]
    tx1 = [tvs[g][0] for g in range(G)]
    ty1 = [tvs[g][1] for g in range(G)]
    tx2 = [tvs[g][2] for g in range(G)]
    ty2 = [tvs[g][3] for g in range(G)]
    area_t = [(tx2[g] - tx1[g]) * (ty2[g] - ty1[g]) for g in range(G)]

    c_0 = jnp.zeros((L,), jnp.int32)
    c_1 = c_0 + 1
    c_2 = c_0 + 2
    c_3 = c_0 + 3

    def _chan_copy(src_h, nch, chstride, imgoff, dst, ci):
        base = pl.multiple_of(ci * CHUNK, CHUNK)
        prs = []
        for c in range(nch):
            so = pl.multiple_of(c * chstride + imgoff + base, 8)
            prs.append((src_h.at[pl.ds(so, CHUNK)],
                        dst.at[pl.ds(c * CHUNK, CHUNK)]))
        for s, d in prs:
            pltpu.async_copy(s, d, dsem)
        for s, d in prs:
            pltpu.make_async_copy(s, d, dsem).wait()

    def p1_chunk(ci, carry):
        base = pl.multiple_of(ci * CHUNK, CHUNK)
        _chan_copy(pri_h, 4, P, 0, pbuf, ci)

        def p1_grp(gi, carry):
            bestv = list(carry[0:G])
            besti = list(carry[G:2 * G])
            idxg = gi * L + idx16
            px = plsc.load_gather(pbuf, [idxg])
            py = plsc.load_gather(pbuf, [idxg + CHUNK])
            pw = plsc.load_gather(pbuf, [idxg + 2 * CHUNK])
            ph = plsc.load_gather(pbuf, [idxg + 3 * CHUNK])
            x1 = px - pw * 0.5
            y1 = py - ph * 0.5
            x2 = px + pw * 0.5
            y2 = py + ph * 0.5
            area_p = pw * ph
            start = base + gi * L
            pidx = start + idx16
            btov = None
            btiv = None
            for g in range(G):
                ix = jnp.minimum(x2, tx2[g]) - jnp.maximum(x1, tx1[g])
                iy = jnp.minimum(y2, ty2[g]) - jnp.maximum(y1, ty1[g])
                inter = jnp.maximum(ix, 0.0) * jnp.maximum(iy, 0.0)
                iou = inter / (area_t[g] + area_p - inter)
                if g == 0:
                    btov = iou
                    btiv = jnp.zeros((L,), jnp.int32)
                else:
                    upd = iou > btov
                    btov = jnp.where(upd, iou, btov)
                    btiv = jnp.where(upd, g, btiv)
                gupd = iou > bestv[g]
                bestv[g] = jnp.where(gupd, iou, bestv[g])
                besti[g] = jnp.where(gupd, pidx, besti[g])
            plsc.store_scatter(bto, [pidx], btov)
            plsc.store_scatter(bti, [pidx], btiv)
            return tuple(bestv) + tuple(besti)

        return lax.fori_loop(0, GROUPS, p1_grp, carry)

    init = tuple(jnp.full((L,), -1.0, jnp.float32) for _ in range(G)) + \
           tuple(jnp.zeros((L,), jnp.int32) for _ in range(G))
    carry = lax.fori_loop(0, NCHUNK, p1_chunk, init)
    bestv = carry[0:G]
    besti = carry[G:2 * G]

    bpo = []
    bpi = []
    for g in range(G):
        m = jnp.max(bestv[g])
        cand = jnp.where(bestv[g] == m, besti[g], jnp.int32(P))
        bpo.append(m)
        bpi.append(jnp.min(cand))
    valid = [bpo[g] >= 0.2 for g in range(G)]
    av = valid[0]
    for g in range(1, G):
        av = jnp.logical_or(av, valid[g])

    def p2_chunk(ci, carry):
        base = pl.multiple_of(ci * CHUNK, CHUNK)
        boff = b * P
        _chan_copy(pri_h, 4, P, 0, pbuf, ci)
        _chan_copy(cls_h, 2, B * P, boff, cbuf, ci)
        _chan_copy(loc_h, 4, B * P, boff, lbuf, ci)
        _chan_copy(lm_h, 10, B * P, boff, mbuf, ci)

        def p2_grp(gi, carry):
            acc_l, acc_c, acc_m, acc_p = carry
            idxg = gi * L + idx16
            start = base + gi * L
            pidx = start + idx16
            px = plsc.load_gather(pbuf, [idxg])
            py = plsc.load_gather(pbuf, [idxg + CHUNK])
            pw = plsc.load_gather(pbuf, [idxg + 2 * CHUNK])
            ph = plsc.load_gather(pbuf, [idxg + 3 * CHUNK])

            orig = plsc.load_gather(bto, [pidx])
            cur = orig
            curi = plsc.load_gather(bti, [pidx])
            for g in range(G):
                hit = pidx == bpi[g]
                upd = jnp.where(valid[g], jnp.float32(2.0), orig)
                cur = jnp.where(hit, upd, cur)
                curi = jnp.where(hit, jnp.int32(g), curi)

            glab = plsc.load_gather(tbuf, [curi, c_0 + 14])
            pos = jnp.logical_and(cur >= THRESHOLD, glab.astype(jnp.int32) != 0)
            pos = jnp.logical_and(pos, av)
            acc_p = acc_p + jnp.where(pos, 1.0, 0.0)

            mx1 = plsc.load_gather(tbuf, [curi, c_0])
            my1 = plsc.load_gather(tbuf, [curi, c_1])
            mx2 = plsc.load_gather(tbuf, [curi, c_2])
            my2 = plsc.load_gather(tbuf, [curi, c_3])
            g_cx = ((mx1 + mx2) * 0.5 - px) / pw * VAR0_INV
            g_cy = ((my1 + my2) * 0.5 - py) / ph * VAR0_INV
            g_w = _vlog((mx2 - mx1) / pw) * VAR1_INV
            g_h = _vlog((my2 - my1) / ph) * VAR1_INV
            l0 = plsc.load_gather(lbuf, [idxg])
            l1 = plsc.load_gather(lbuf, [idxg + CHUNK])
            l2 = plsc.load_gather(lbuf, [idxg + 2 * CHUNK])
            l3 = plsc.load_gather(lbuf, [idxg + 3 * CHUNK])
            vloc = _sl1(l0, g_cx) + _sl1(l1, g_cy) + _sl1(l2, g_w) + _sl1(l3, g_h)
            acc_l = acc_l + jnp.where(pos, vloc, 0.0)

            vlm = None
            for j in range(5):
                for xy in range(2):
                    ch = 2 * j + xy
                    mv = plsc.load_gather(tbuf, [curi, c_0 + (4 + ch)])
                    pc = px if xy == 0 else py
                    ps = pw if xy == 0 else ph
                    tv = (mv - pc) / ps * VAR0_INV
                    dat = plsc.load_gather(mbuf, [idxg + ch * CHUNK])
                    t = _sl1(dat, tv)
                    vlm = t if vlm is None else vlm + t
            acc_m = acc_m + jnp.where(pos, vlm, 0.0)

            c0 = plsc.load_gather(cbuf, [idxg])
            c1 = plsc.load_gather(cbuf, [idxg + CHUNK])
            mx = jnp.maximum(c0, c1)
            lse = mx + _vlog1p(jnp.exp(-jnp.abs(c0 - c1)))
            acc_c = acc_c + jnp.where(pos, lse - c1, 0.0)
            plsc.store_scatter(lcb, [pidx], jnp.where(pos, 0.0, lse - c0))
            return acc_l, acc_c, acc_m, acc_p

        return lax.fori_loop(0, GROUPS, p2_grp, carry)

    zf = jnp.zeros((L,), jnp.float32)
    acc_l, acc_c, acc_m, acc_p = lax.fori_loop(0, NCHUNK, p2_chunk, (zf, zf, zf, zf))

    npos = jnp.sum(acc_p)
    k = jnp.minimum(npos.astype(jnp.int32) * NEGPOS_RATIO, P - 1)

    ones_i = jnp.ones((L,), jnp.int32)
    prefix = jnp.int32(0)
    k_cur = k
    for shift, nbits in ((23, 8), (15, 8), (7, 8), (0, 7)):
        himask = jnp.int32(-(1 << (shift + nbits)))

        def h_zero(gi, _):
            plsc.store_scatter(hist, [gi * L + idx16], jnp.zeros((L,), jnp.int32))
            return 0

        lax.fori_loop(0, 256 // L, h_zero, 0)

        def h_grp(gi, _):
            bits = plsc.bitcast(plsc.load_gather(lcb, [gi * L + idx16]), jnp.int32)
            match = (bits & himask) == prefix
            bucket = (bits >> shift) & ((1 << nbits) - 1)
            plsc.addupdate_scatter(hist, [bucket], ones_i, mask=match)
            return 0

        lax.fori_loop(0, P // L, h_grp, 0)

        bstar = jnp.int32(-1)
        carry_cnt = jnp.int32(0)
        for gi in reversed(range(256 // L)):
            h = hist[pl.ds(gi * L, L)]
            sfx = lax.rev(plsc.cumsum(lax.rev(h, (0,))), (0,))
            ge = sfx + carry_cnt
            gidx = gi * L + idx16
            cand = jnp.where(ge >= k_cur, gidx, jnp.int32(-1))
            bstar = jnp.maximum(bstar, jnp.max(cand))
            carry_cnt = carry_cnt + jnp.max(sfx)
        cgt = jnp.int32(0)
        for gi in range(256 // L):
            h = hist[pl.ds(gi * L, L)]
            gidx = gi * L + idx16
            cgt = cgt + jnp.sum(jnp.where(gidx > bstar, h, 0))
        k_cur = k_cur - cgt
        prefix = prefix | jnp.where(bstar > 0, bstar, 0) << shift

    t_bits = prefix

    def f_grp(gi, carry):
        cgt_a, s_a = carry
        v = plsc.load_gather(lcb, [gi * L + idx16])
        bits = plsc.bitcast(v, jnp.int32)
        gt = bits > t_bits
        return cgt_a + jnp.where(gt, 1, 0), s_a + jnp.where(gt, v, 0.0)

    cgt_a, s_a = lax.fori_loop(0, P // L, f_grp,
                               (jnp.zeros((L,), jnp.int32), zf))
    cgt = jnp.sum(cgt_a)
    s_gt = jnp.sum(s_a)
    t_val = lax.bitcast_convert_type(t_bits, jnp.float32)
    topk = s_gt + (k - cgt).astype(jnp.float32) * t_val
    topk = jnp.where(k > 0, topk, 0.0)

    sum_l = jnp.sum(acc_l)
    sum_c = jnp.sum(acc_c) + topk
    sum_m = jnp.sum(acc_m)
    res = jnp.where(idx16 == 0, sum_l, 0.0)
    res = jnp.where(idx16 == 1, sum_c, res)
    res = jnp.where(idx16 == 2, sum_m, res)
    res = jnp.where(idx16 == 3, npos, res)
    obuf[...] = res
    pltpu.sync_copy(obuf, out_h.at[pl.ds(pl.multiple_of(wid * L, 8), L)])


@functools.partial(jax.jit, static_argnames=())
def _run(cls_f, loc_f, lm_f, pri_f, tgt_f):
    mesh = plsc.VectorSubcoreMesh(core_axis_name="c", subcore_axis_name="s",
                                  num_cores=2, num_subcores=16)
    f = pl.kernel(
        _body,
        out_type=jax.ShapeDtypeStruct((B * L,), jnp.float32),
        mesh=mesh,
        scratch_types=[
            pltpu.VMEM((CHUNK * 4,), jnp.float32),
            pltpu.VMEM((CHUNK * 2,), jnp.float32),
            pltpu.VMEM((CHUNK * 4,), jnp.float32),
            pltpu.VMEM((CHUNK * 10,), jnp.float32),
            pltpu.VMEM((G, TGT_STRIDE), jnp.float32),
            pltpu.VMEM((P,), jnp.float32),
            pltpu.VMEM((P,), jnp.int32),
            pltpu.VMEM((P,), jnp.float32),
            pltpu.VMEM((256,), jnp.int32),
            pltpu.VMEM((L,), jnp.float32),
            pltpu.SemaphoreType.DMA,
        ],
        compiler_params=pltpu.CompilerParams(needs_layout_passes=False,
                                             use_tc_tiling_on_sc=False),
    )
    return f(cls_f, loc_f, lm_f, pri_f, tgt_f)


def kernel(cls_data, loc_data, landm_data, priors, targets):
    cls_t = cls_data.transpose(2, 0, 1).reshape(2 * B * P)
    loc_t = loc_data.transpose(2, 0, 1).reshape(4 * B * P)
    lm_t = landm_data.transpose(2, 0, 1).reshape(10 * B * P)
    pri_t = priors.transpose(1, 0).reshape(4 * P)
    out = _run(cls_t, loc_t, lm_t, pri_t, targets).reshape(B, L)
    npos = jnp.sum(out[:, 3])
    n = jnp.maximum(npos, 1.0)
    return jnp.sum(out[:, 0]) / n, jnp.sum(out[:, 1]) / n, jnp.sum(out[:, 2]) / n


# speedup vs baseline: 4.4518x; 1.2159x over previous
import functools

import jax
import jax.numpy as jnp
from jax import lax
from jax.experimental import pallas as pl
from jax.experimental.pallas import tpu as pltpu
from jax.experimental.pallas import tpu_sc as plsc

B = 32
P = 16800
G = 8
L = 16
CHUNK = 1680
NCHUNK = P // CHUNK
GROUPS = CHUNK // L
TGT_STRIDE = 15
THRESHOLD = 0.35
NEGPOS_RATIO = 7
VAR0_INV = 10.0
VAR1_INV = 5.0
LN2 = 0.6931471805599453


def _vlog(x):
    bits = plsc.bitcast(x, jnp.int32)
    e = (bits >> 23) - 127
    m = plsc.bitcast((bits & 0x7FFFFF) | (127 << 23), jnp.float32)
    big = m > 1.4142135623730951
    m = jnp.where(big, m * 0.5, m)
    e = jnp.where(big, e + 1, e)
    s = (m - 1.0) / (m + 1.0)
    z = s * s
    p = 2.0 * s * (1.0 + z * (1.0 / 3.0 + z * (0.2 + z * (1.0 / 7.0 + z * (1.0 / 9.0)))))
    return e.astype(jnp.float32) * LN2 + p


def _vlog1p(e):
    u = e / (e + 2.0)
    z = u * u
    return 2.0 * u * (1.0 + z * (1.0 / 3.0 + z * (0.2 + z * (1.0 / 7.0
                      + z * (1.0 / 9.0 + z * (1.0 / 11.0))))))


def _sl1(a, b):
    d = jnp.abs(a - b)
    return jnp.where(d < 1.0, 0.5 * d * d, d - 0.5)


def _body(cls_h, loc_h, lm_h, pri_h, tgt_h, out_h,
          pbuf, cbuf, lbuf, mbuf, tbuf, bto, bti, lcb, hist, obuf,
          dsem0, dsem1):
    cid = lax.axis_index("c")
    sid = lax.axis_index("s")
    wid = sid * 2 + cid
    b = wid

    pltpu.sync_copy(tgt_h.at[b], tbuf)

    idx16 = lax.iota(jnp.int32, 16)
    cl14 = jnp.minimum(idx16, 14)

    tvs = [plsc.load_gather(tbuf, [jnp.full((L,), g, jnp.int32), cl14])
           for g in range(G)]
    tx1 = [tvs[g][0] for g in range(G)]
    ty1 = [tvs[g][1] for g in range(G)]
    tx2 = [tvs[g][2] for g in range(G)]
    ty2 = [tvs[g][3] for g in range(G)]
    area_t = [(tx2[g] - tx1[g]) * (ty2[g] - ty1[g]) for g in range(G)]

    c_0 = jnp.zeros((L,), jnp.int32)
    c_1 = c_0 + 1
    c_2 = c_0 + 2
    c_3 = c_0 + 3

    sems = (dsem0, dsem1)

    def _chan_pairs(src_h, nch, chstride, imgoff, dst, doff, ci):
        base = pl.multiple_of(ci * CHUNK, CHUNK)
        prs = []
        for c in range(nch):
            so = pl.multiple_of(c * chstride + imgoff + base, 8)
            prs.append((src_h.at[pl.ds(so, CHUNK)],
                        dst.at[pl.ds(doff + c * CHUNK, CHUNK)]))
        return prs

    def _start(prs, par):
        for s, d in prs:
            pltpu.async_copy(s, d, sems[par])

    def _drain(prs, par):
        for s, d in prs:
            pltpu.make_async_copy(s, d, sems[par]).wait()

    def p1_pairs(ci, par):
        return _chan_pairs(pri_h, 4, P, 0, pbuf, par * (4 * CHUNK), ci)

    def p1_chunk(ci, par, carry):
        base = pl.multiple_of(ci * CHUNK, CHUNK)
        pb = par * (4 * CHUNK)

        def p1_grp(gi, carry):
            bestv = list(carry[0:G])
            besti = list(carry[G:2 * G])
            idxg = pb + gi * L + idx16
            px = plsc.load_gather(pbuf, [idxg])
            py = plsc.load_gather(pbuf, [idxg + CHUNK])
            pw = plsc.load_gather(pbuf, [idxg + 2 * CHUNK])
            ph = plsc.load_gather(pbuf, [idxg + 3 * CHUNK])
            x1 = px - pw * 0.5
            y1 = py - ph * 0.5
            x2 = px + pw * 0.5
            y2 = py + ph * 0.5
            area_p = pw * ph
            start = base + gi * L
            pidx = start + idx16
            btov = None
            btiv = None
            for g in range(G):
                ix = jnp.minimum(x2, tx2[g]) - jnp.maximum(x1, tx1[g])
                iy = jnp.minimum(y2, ty2[g]) - jnp.maximum(y1, ty1[g])
                inter = jnp.maximum(ix, 0.0) * jnp.maximum(iy, 0.0)
                iou = inter / (area_t[g] + area_p - inter)
                if g == 0:
                    btov = iou
                    btiv = jnp.zeros((L,), jnp.int32)
                else:
                    upd = iou > btov
                    btov = jnp.where(upd, iou, btov)
                    btiv = jnp.where(upd, g, btiv)
                gupd = iou > bestv[g]
                bestv[g] = jnp.where(gupd, iou, bestv[g])
                besti[g] = jnp.where(gupd, pidx, besti[g])
            plsc.store_scatter(bto, [pidx], btov)
            plsc.store_scatter(bti, [pidx], btiv)
            return tuple(bestv) + tuple(besti)

        return lax.fori_loop(0, GROUPS, p1_grp, carry)

    init = tuple(jnp.full((L,), -1.0, jnp.float32) for _ in range(G)) + \
           tuple(jnp.zeros((L,), jnp.int32) for _ in range(G))
    _start(p1_pairs(0, 0), 0)

    def p1_outer(gg, carry):
        for par in (0, 1):
            ci = gg * 2 + par
            _drain(p1_pairs(ci, par), par)

            @pl.when(ci + 1 < NCHUNK)
            def _():
                _start(p1_pairs(ci + 1, 1 - par), 1 - par)

            carry = p1_chunk(ci, par, carry)
        return carry

    carry = lax.fori_loop(0, NCHUNK // 2, p1_outer, init)
    bestv = carry[0:G]
    besti = carry[G:2 * G]

    bpo = []
    bpi = []
    for g in range(G):
        m = jnp.max(bestv[g])
        cand = jnp.where(bestv[g] == m, besti[g], jnp.int32(P))
        bpo.append(m)
        bpi.append(jnp.min(cand))
    valid = [bpo[g] >= 0.2 for g in range(G)]
    av = valid[0]
    for g in range(1, G):
        av = jnp.logical_or(av, valid[g])

    boff = b * P

    def p2_pairs(ci, par):
        return (_chan_pairs(pri_h, 4, P, 0, pbuf, par * (4 * CHUNK), ci)
                + _chan_pairs(cls_h, 2, B * P, boff, cbuf, par * (2 * CHUNK), ci)
                + _chan_pairs(loc_h, 4, B * P, boff, lbuf, par * (4 * CHUNK), ci)
                + _chan_pairs(lm_h, 10, B * P, boff, mbuf, par * (10 * CHUNK), ci))

    def p2_chunk(ci, par, carry):
        base = pl.multiple_of(ci * CHUNK, CHUNK)
        pb4 = par * (4 * CHUNK)
        pb2 = par * (2 * CHUNK)
        pb10 = par * (10 * CHUNK)

        def p2_grp(gi, carry):
            acc_l, acc_c, acc_m, acc_p = carry
            idxg = pb4 + gi * L + idx16
            idxg2 = pb2 + gi * L + idx16
            idxg10 = pb10 + gi * L + idx16
            start = base + gi * L
            pidx = start + idx16
            px = plsc.load_gather(pbuf, [idxg])
            py = plsc.load_gather(pbuf, [idxg + CHUNK])
            pw = plsc.load_gather(pbuf, [idxg + 2 * CHUNK])
            ph = plsc.load_gather(pbuf, [idxg + 3 * CHUNK])

            orig = plsc.load_gather(bto, [pidx])
            cur = orig
            curi = plsc.load_gather(bti, [pidx])
            for g in range(G):
                hit = pidx == bpi[g]
                upd = jnp.where(valid[g], jnp.float32(2.0), orig)
                cur = jnp.where(hit, upd, cur)
                curi = jnp.where(hit, jnp.int32(g), curi)

            glab = plsc.load_gather(tbuf, [curi, c_0 + 14])
            pos = jnp.logical_and(cur >= THRESHOLD, glab.astype(jnp.int32) != 0)
            pos = jnp.logical_and(pos, av)
            acc_p = acc_p + jnp.where(pos, 1.0, 0.0)

            mx1 = plsc.load_gather(tbuf, [curi, c_0])
            my1 = plsc.load_gather(tbuf, [curi, c_1])
            mx2 = plsc.load_gather(tbuf, [curi, c_2])
            my2 = plsc.load_gather(tbuf, [curi, c_3])
            g_cx = ((mx1 + mx2) * 0.5 - px) / pw * VAR0_INV
            g_cy = ((my1 + my2) * 0.5 - py) / ph * VAR0_INV
            g_w = _vlog((mx2 - mx1) / pw) * VAR1_INV
            g_h = _vlog((my2 - my1) / ph) * VAR1_INV
            l0 = plsc.load_gather(lbuf, [idxg])
            l1 = plsc.load_gather(lbuf, [idxg + CHUNK])
            l2 = plsc.load_gather(lbuf, [idxg + 2 * CHUNK])
            l3 = plsc.load_gather(lbuf, [idxg + 3 * CHUNK])
            vloc = _sl1(l0, g_cx) + _sl1(l1, g_cy) + _sl1(l2, g_w) + _sl1(l3, g_h)
            acc_l = acc_l + jnp.where(pos, vloc, 0.0)

            vlm = None
            for j in range(5):
                for xy in range(2):
                    ch = 2 * j + xy
                    mv = plsc.load_gather(tbuf, [curi, c_0 + (4 + ch)])
                    pc = px if xy == 0 else py
                    ps = pw if xy == 0 else ph
                    tv = (mv - pc) / ps * VAR0_INV
                    dat = plsc.load_gather(mbuf, [idxg10 + ch * CHUNK])
                    t = _sl1(dat, tv)
                    vlm = t if vlm is None else vlm + t
            acc_m = acc_m + jnp.where(pos, vlm, 0.0)

            c0 = plsc.load_gather(cbuf, [idxg2])
            c1 = plsc.load_gather(cbuf, [idxg2 + CHUNK])
            mx = jnp.maximum(c0, c1)
            lse = mx + _vlog1p(jnp.exp(-jnp.abs(c0 - c1)))
            acc_c = acc_c + jnp.where(pos, lse - c1, 0.0)
            plsc.store_scatter(lcb, [pidx], jnp.where(pos, 0.0, lse - c0))
            return acc_l, acc_c, acc_m, acc_p

        return lax.fori_loop(0, GROUPS, p2_grp, carry)

    zf = jnp.zeros((L,), jnp.float32)
    _start(p2_pairs(0, 0), 0)

    def p2_outer(gg, carry):
        for par in (0, 1):
            ci = gg * 2 + par
            _drain(p2_pairs(ci, par), par)

            @pl.when(ci + 1 < NCHUNK)
            def _():
                _start(p2_pairs(ci + 1, 1 - par), 1 - par)

            carry = p2_chunk(ci, par, carry)
        return carry

    acc_l, acc_c, acc_m, acc_p = lax.fori_loop(0, NCHUNK // 2, p2_outer,
                                               (zf, zf, zf, zf))

    npos = jnp.sum(acc_p)
    k = jnp.minimum(npos.astype(jnp.int32) * NEGPOS_RATIO, P - 1)

    ones_i = jnp.ones((L,), jnp.int32)
    prefix = jnp.int32(0)
    k_cur = k
    for shift, nbits in ((23, 8), (15, 8), (7, 8), (0, 7)):
        himask = jnp.int32(-(1 << (shift + nbits)))

        def h_zero(gi, _):
            plsc.store_scatter(hist, [gi * L + idx16], jnp.zeros((L,), jnp.int32))
            return 0

        lax.fori_loop(0, 256 // L, h_zero, 0)

        def h_grp(gi, _):
            base5 = gi * (5 * L)
            bs = [plsc.bitcast(plsc.load_gather(lcb, [base5 + u * L + idx16]),
                               jnp.int32) for u in range(5)]
            for u in range(5):
                match = (bs[u] & himask) == prefix
                bucket = (bs[u] >> shift) & ((1 << nbits) - 1)
                plsc.addupdate_scatter(hist, [bucket], ones_i, mask=match)
            return 0

        lax.fori_loop(0, P // L // 5, h_grp, 0)

        bstar = jnp.int32(-1)
        carry_cnt = jnp.int32(0)
        for gi in reversed(range(256 // L)):
            h = hist[pl.ds(gi * L, L)]
            sfx = lax.rev(plsc.cumsum(lax.rev(h, (0,))), (0,))
            ge = sfx + carry_cnt
            gidx = gi * L + idx16
            cand = jnp.where(ge >= k_cur, gidx, jnp.int32(-1))
            bstar = jnp.maximum(bstar, jnp.max(cand))
            carry_cnt = carry_cnt + jnp.max(sfx)
        cgt = jnp.int32(0)
        for gi in range(256 // L):
            h = hist[pl.ds(gi * L, L)]
            gidx = gi * L + idx16
            cgt = cgt + jnp.sum(jnp.where(gidx > bstar, h, 0))
        k_cur = k_cur - cgt
        prefix = prefix | jnp.where(bstar > 0, bstar, 0) << shift

    t_bits = prefix

    def f_grp(gi, carry):
        cgt_a, s_a = carry
        base5 = gi * (5 * L)
        for u in range(5):
            v = plsc.load_gather(lcb, [base5 + u * L + idx16])
            bits = plsc.bitcast(v, jnp.int32)
            gt = bits > t_bits
            cgt_a = cgt_a + jnp.where(gt, 1, 0)
            s_a = s_a + jnp.where(gt, v, 0.0)
        return cgt_a, s_a

    cgt_a, s_a = lax.fori_loop(0, P // L // 5, f_grp,
                               (jnp.zeros((L,), jnp.int32), zf))
    cgt = jnp.sum(cgt_a)
    s_gt = jnp.sum(s_a)
    t_val = lax.bitcast_convert_type(t_bits, jnp.float32)
    topk = s_gt + (k - cgt).astype(jnp.float32) * t_val
    topk = jnp.where(k > 0, topk, 0.0)

    sum_l = jnp.sum(acc_l)
    sum_c = jnp.sum(acc_c) + topk
    sum_m = jnp.sum(acc_m)
    res = jnp.where(idx16 == 0, sum_l, 0.0)
    res = jnp.where(idx16 == 1, sum_c, res)
    res = jnp.where(idx16 == 2, sum_m, res)
    res = jnp.where(idx16 == 3, npos, res)
    obuf[...] = res
    pltpu.sync_copy(obuf, out_h.at[pl.ds(pl.multiple_of(wid * L, 8), L)])


@functools.partial(jax.jit, static_argnames=())
def _run(cls_f, loc_f, lm_f, pri_f, tgt_f):
    mesh = plsc.VectorSubcoreMesh(core_axis_name="c", subcore_axis_name="s",
                                  num_cores=2, num_subcores=16)
    f = pl.kernel(
        _body,
        out_type=jax.ShapeDtypeStruct((B * L,), jnp.float32),
        mesh=mesh,
        scratch_types=[
            pltpu.VMEM((2 * CHUNK * 4,), jnp.float32),
            pltpu.VMEM((2 * CHUNK * 2,), jnp.float32),
            pltpu.VMEM((2 * CHUNK * 4,), jnp.float32),
            pltpu.VMEM((2 * CHUNK * 10,), jnp.float32),
            pltpu.VMEM((G, TGT_STRIDE), jnp.float32),
            pltpu.VMEM((P,), jnp.float32),
            pltpu.VMEM((P,), jnp.int32),
            pltpu.VMEM((P,), jnp.float32),
            pltpu.VMEM((256,), jnp.int32),
            pltpu.VMEM((L,), jnp.float32),
            pltpu.SemaphoreType.DMA,
            pltpu.SemaphoreType.DMA,
        ],
        compiler_params=pltpu.CompilerParams(needs_layout_passes=False,
                                             use_tc_tiling_on_sc=False),
    )
    return f(cls_f, loc_f, lm_f, pri_f, tgt_f)


def kernel(cls_data, loc_data, landm_data, priors, targets):
    cls_t = cls_data.transpose(2, 0, 1).reshape(2 * B * P)
    loc_t = loc_data.transpose(2, 0, 1).reshape(4 * B * P)
    lm_t = landm_data.transpose(2, 0, 1).reshape(10 * B * P)
    pri_t = priors.transpose(1, 0).reshape(4 * P)
    out = _run(cls_t, loc_t, lm_t, pri_t, targets).reshape(B, L)
    npos = jnp.sum(out[:, 3])
    n = jnp.maximum(npos, 1.0)
    return jnp.sum(out[:, 0]) / n, jnp.sum(out[:, 1]) / n, jnp.sum(out[:, 2]) / n
